# baseline (speedup 1.0000x reference)
import functools

import jax
import jax.numpy as jnp
from jax import lax
from jax.experimental import pallas as pl
from jax.experimental.pallas import tpu as pltpu

_RADIUS_SQ = 64.0
_TJ = 256


def _dot(a, b, dims, precision):
    return lax.dot_general(a, b, (dims, ((), ())),
                           preferred_element_type=jnp.float32,
                           precision=precision)


def _body(x_ref, p_ref, g_ref, WgT_ref, bg_ref, WlT_ref, bl_ref,
          pos_out_ref, feat_out_ref, g_out_ref, *, num_pos, num_glob):
    for c in range(x_ref.shape[0]):
        _one_cloud(x_ref[c], p_ref[c], g_ref[c], WgT_ref[...], bg_ref[...],
                   WlT_ref[...], bl_ref[...], pos_out_ref.at[c],
                   feat_out_ref.at[c], g_out_ref.at[c], num_pos, num_glob)


def _one_cloud(x, p, g_col, WgT, bg, WlT, bl,
               pos_out_ref, feat_out_ref, g_out_ref, num_pos, num_glob):
    P, G = num_pos, num_glob
    C, N = x.shape
    TJ = _TJ if N % _TJ == 0 else N
    NC = N // TJ

    x_aug = jnp.concatenate(
        [x, jnp.ones((1, N), jnp.float32)], axis=0)
    x_aug_bf = x_aug.astype(jnp.bfloat16)
    n_row = jnp.sum(p * p, axis=0, keepdims=True)
    ones_row = jnp.ones((1, N), jnp.float32)
    c_row = 0.25 * _RADIUS_SQ - 0.5 * n_row
    u = jnp.concatenate([p, c_row, ones_row], axis=0)
    v = jnp.concatenate([p, ones_row, c_row], axis=0)
    u_hi = u.astype(jnp.bfloat16)
    u_lo = (u - u_hi.astype(jnp.float32)).astype(jnp.bfloat16)
    v_hi = v.astype(jnp.bfloat16)
    v_lo = (v - v_hi.astype(jnp.float32)).astype(jnp.bfloat16)
    u_stack = jnp.concatenate([u_hi, u_hi, u_lo], axis=0)
    v_stack = jnp.concatenate([v_hi, v_lo, v_hi], axis=0)

    s_chunks = [jnp.zeros((C + 1, TJ), jnp.float32) for _ in range(NC)]
    for j in range(NC):
        slj = slice(j * TJ, (j + 1) * TJ)
        rows = (j + 1) * TJ
        s_j = _dot(u_stack[:, :rows], v_stack[:, slj], ((0,), (0,)),
                   lax.Precision.DEFAULT)
        mask = (s_j > 0.0).astype(jnp.float32).astype(jnp.bfloat16)
        contrib = _dot(x_aug_bf[:, slj], mask, ((1,), (1,)),
                       lax.Precision.DEFAULT)
        for i in range(j + 1):
            s_chunks[i] = s_chunks[i] + contrib[:, i * TJ:(i + 1) * TJ]
        if j > 0:
            s_chunks[j] = s_chunks[j] + _dot(
                x_aug_bf[:, :j * TJ], mask[:j * TJ], ((1,), (0,)),
                lax.Precision.DEFAULT)
    s = jnp.concatenate(s_chunks, axis=1)
    counts = jnp.maximum(s[C:], 1.0)
    neigh_mean = s[:C] / counts

    agg = jnp.max(x, axis=1, keepdims=True)
    g_in = jnp.concatenate([agg, g_col], axis=0)
    g_out = jnp.tanh(
        _dot(WgT, g_in, ((1,), (0,)), lax.Precision.HIGHEST)
        + bg)
    g_out_ref[...] = g_out
    gu = g_out[G:]

    local_in = jnp.concatenate(
        [x, neigh_mean, jnp.broadcast_to(gu, (G, N))], axis=0)
    li_hi = local_in.astype(jnp.bfloat16)
    li_lo = (local_in - li_hi.astype(jnp.float32)).astype(jnp.bfloat16)
    li_stack = jnp.concatenate([li_hi, li_lo, li_hi], axis=0)
    W = WlT
    W_hi = W.astype(jnp.bfloat16)
    W_lo = (W - W_hi.astype(jnp.float32)).astype(jnp.bfloat16)
    W_stack = jnp.concatenate([W_hi, W_hi, W_lo], axis=1)
    lo = jnp.tanh(
        _dot(W_stack, li_stack, ((1,), (0,)), lax.Precision.DEFAULT)
        + bl)
    pos_out_ref[...] = lo[:P]
    feat_out_ref[...] = lo[P:]


def kernel(positions, features, global_features, W_g, b_g, W_l, b_l):
    B, P, N = positions.shape
    F = features.shape[1]
    C = P + F
    G = global_features.shape[1]

    x = jnp.concatenate([positions, features], axis=1)
    g3 = global_features[:, :, None]
    WgT = W_g.T
    bg = b_g[:, None]
    WlT = W_l.T
    bl = b_l[:, None]

    CB = 2 if B % 2 == 0 else 1
    pos_new, feat_new, gout = pl.pallas_call(
        functools.partial(_body, num_pos=P, num_glob=G),
        grid=(B // CB,),
        compiler_params=pltpu.CompilerParams(
            dimension_semantics=("parallel",)),
        in_specs=[
            pl.BlockSpec((CB, C, N), lambda b: (b, 0, 0)),
            pl.BlockSpec((CB, P, N), lambda b: (b, 0, 0)),
            pl.BlockSpec((CB, G, 1), lambda b: (b, 0, 0)),
            pl.BlockSpec((2 * G, C + G), lambda b: (0, 0)),
            pl.BlockSpec((2 * G, 1), lambda b: (0, 0)),
            pl.BlockSpec((C, 2 * C + G), lambda b: (0, 0)),
            pl.BlockSpec((C, 1), lambda b: (0, 0)),
        ],
        out_specs=[
            pl.BlockSpec((CB, P, N), lambda b: (b, 0, 0)),
            pl.BlockSpec((CB, F, N), lambda b: (b, 0, 0)),
            pl.BlockSpec((CB, 2 * G, 1), lambda b: (b, 0, 0)),
        ],
        out_shape=[
            jax.ShapeDtypeStruct((B, P, N), jnp.float32),
            jax.ShapeDtypeStruct((B, F, N), jnp.float32),
            jax.ShapeDtypeStruct((B, 2 * G, 1), jnp.float32),
        ],
    )(x, positions, g3, WgT, bg, WlT, bl)

    return (pos_new, feat_new, gout[:, :G, 0])


# speedup vs baseline: 1.1163x; 1.1163x over previous
import functools

import jax
import jax.numpy as jnp
from jax import lax
from jax.experimental import pallas as pl
from jax.experimental.pallas import tpu as pltpu

_RADIUS_SQ = 64.0
_TJ = 512


def _dot(a, b, dims, precision):
    return lax.dot_general(a, b, (dims, ((), ())),
                           preferred_element_type=jnp.float32,
                           precision=precision)


def _body(x_ref, p_ref, g_ref, WgT_ref, bg_ref, WlT_ref, bl_ref,
          pos_out_ref, feat_out_ref, g_out_ref, *, num_pos, num_glob):
    for c in range(x_ref.shape[0]):
        _one_cloud(x_ref[c], p_ref[c], g_ref[c], WgT_ref[...], bg_ref[...],
                   WlT_ref[...], bl_ref[...], pos_out_ref.at[c],
                   feat_out_ref.at[c], g_out_ref.at[c], num_pos, num_glob)


def _one_cloud(x, p, g_col, WgT, bg, WlT, bl,
               pos_out_ref, feat_out_ref, g_out_ref, num_pos, num_glob):
    P, G = num_pos, num_glob
    C, N = x.shape
    TJ = _TJ if N % _TJ == 0 else N
    NC = N // TJ

    x_aug = jnp.concatenate(
        [x, jnp.ones((1, N), jnp.float32)], axis=0)
    x_aug_bf = x_aug.astype(jnp.bfloat16)
    n_row = jnp.sum(p * p, axis=0, keepdims=True)
    ones_row = jnp.ones((1, N), jnp.float32)
    c_row = 0.25 * _RADIUS_SQ - 0.5 * n_row
    u = jnp.concatenate([p, c_row, ones_row], axis=0)
    v = jnp.concatenate([p, ones_row, c_row], axis=0)
    u_hi = u.astype(jnp.bfloat16)
    u_lo = (u - u_hi.astype(jnp.float32)).astype(jnp.bfloat16)
    v_hi = v.astype(jnp.bfloat16)
    v_lo = (v - v_hi.astype(jnp.float32)).astype(jnp.bfloat16)
    u_stack = jnp.concatenate([u_hi, u_hi, u_lo], axis=0)
    v_stack = jnp.concatenate([v_hi, v_lo, v_hi], axis=0)

    s_chunks = [jnp.zeros((C + 1, TJ), jnp.float32) for _ in range(NC)]
    for j in range(NC):
        slj = slice(j * TJ, (j + 1) * TJ)
        rows = (j + 1) * TJ
        s_j = _dot(u_stack[:, :rows], v_stack[:, slj], ((0,), (0,)),
                   lax.Precision.DEFAULT)
        mask = (s_j > 0.0).astype(jnp.float32).astype(jnp.bfloat16)
        contrib = _dot(x_aug_bf[:, slj], mask, ((1,), (1,)),
                       lax.Precision.DEFAULT)
        for i in range(j + 1):
            s_chunks[i] = s_chunks[i] + contrib[:, i * TJ:(i + 1) * TJ]
        if j > 0:
            s_chunks[j] = s_chunks[j] + _dot(
                x_aug_bf[:, :j * TJ], mask[:j * TJ], ((1,), (0,)),
                lax.Precision.DEFAULT)
    s = jnp.concatenate(s_chunks, axis=1)
    counts = jnp.maximum(s[C:], 1.0)
    neigh_mean = s[:C] / counts

    agg = jnp.max(x, axis=1, keepdims=True)
    g_in = jnp.concatenate([agg, g_col], axis=0)
    g_out = jnp.tanh(
        _dot(WgT, g_in, ((1,), (0,)), lax.Precision.HIGHEST)
        + bg)
    g_out_ref[...] = g_out
    gu = g_out[G:]

    local_in = jnp.concatenate(
        [x, neigh_mean, jnp.broadcast_to(gu, (G, N))], axis=0)
    li_hi = local_in.astype(jnp.bfloat16)
    li_lo = (local_in - li_hi.astype(jnp.float32)).astype(jnp.bfloat16)
    li_stack = jnp.concatenate([li_hi, li_lo, li_hi], axis=0)
    W = WlT
    W_hi = W.astype(jnp.bfloat16)
    W_lo = (W - W_hi.astype(jnp.float32)).astype(jnp.bfloat16)
    W_stack = jnp.concatenate([W_hi, W_hi, W_lo], axis=1)
    lo = jnp.tanh(
        _dot(W_stack, li_stack, ((1,), (0,)), lax.Precision.DEFAULT)
        + bl)
    pos_out_ref[...] = lo[:P]
    feat_out_ref[...] = lo[P:]


def kernel(positions, features, global_features, W_g, b_g, W_l, b_l):
    B, P, N = positions.shape
    F = features.shape[1]
    C = P + F
    G = global_features.shape[1]

    x = jnp.concatenate([positions, features], axis=1)
    g3 = global_features[:, :, None]
    WgT = W_g.T
    bg = b_g[:, None]
    WlT = W_l.T
    bl = b_l[:, None]

    CB = 2 if B % 2 == 0 else 1
    pos_new, feat_new, gout = pl.pallas_call(
        functools.partial(_body, num_pos=P, num_glob=G),
        grid=(B // CB,),
        compiler_params=pltpu.CompilerParams(
            dimension_semantics=("parallel",)),
        in_specs=[
            pl.BlockSpec((CB, C, N), lambda b: (b, 0, 0)),
            pl.BlockSpec((CB, P, N), lambda b: (b, 0, 0)),
            pl.BlockSpec((CB, G, 1), lambda b: (b, 0, 0)),
            pl.BlockSpec((2 * G, C + G), lambda b: (0, 0)),
            pl.BlockSpec((2 * G, 1), lambda b: (0, 0)),
            pl.BlockSpec((C, 2 * C + G), lambda b: (0, 0)),
            pl.BlockSpec((C, 1), lambda b: (0, 0)),
        ],
        out_specs=[
            pl.BlockSpec((CB, P, N), lambda b: (b, 0, 0)),
            pl.BlockSpec((CB, F, N), lambda b: (b, 0, 0)),
            pl.BlockSpec((CB, 2 * G, 1), lambda b: (b, 0, 0)),
        ],
        out_shape=[
            jax.ShapeDtypeStruct((B, P, N), jnp.float32),
            jax.ShapeDtypeStruct((B, F, N), jnp.float32),
            jax.ShapeDtypeStruct((B, 2 * G, 1), jnp.float32),
        ],
    )(x, positions, g3, WgT, bg, WlT, bl)

    return (pos_new, feat_new, gout[:, :G, 0])


# speedup vs baseline: 1.1650x; 1.0437x over previous
import functools

import jax
import jax.numpy as jnp
from jax import lax
from jax.experimental import pallas as pl
from jax.experimental.pallas import tpu as pltpu

_RADIUS_SQ = 64.0
_TJ = 512


def _dot(a, b, dims, precision):
    return lax.dot_general(a, b, (dims, ((), ())),
                           preferred_element_type=jnp.float32,
                           precision=precision)


def _body(x_ref, p_ref, g_ref, WgT_ref, bg_ref, WlT_ref, bl_ref,
          pos_out_ref, feat_out_ref, g_out_ref, *, num_pos, num_glob):
    for c in range(x_ref.shape[0]):
        _one_cloud(x_ref[c], p_ref[c], g_ref[c], WgT_ref[...], bg_ref[...],
                   WlT_ref[...], bl_ref[...], pos_out_ref.at[c],
                   feat_out_ref.at[c], g_out_ref.at[c], num_pos, num_glob)


def _one_cloud(x, p, g_col, WgT, bg, WlT, bl,
               pos_out_ref, feat_out_ref, g_out_ref, num_pos, num_glob):
    P, G = num_pos, num_glob
    C, N = x.shape
    TJ = _TJ if N % _TJ == 0 else N
    NC = N // TJ

    agg = jnp.max(x, axis=1, keepdims=True)
    g_in = jnp.concatenate([agg, g_col], axis=0)
    g_out = jnp.tanh(
        _dot(WgT, g_in, ((1,), (0,)), lax.Precision.HIGHEST)
        + bg)
    g_out_ref[...] = g_out
    gu = g_out[G:]

    x_aug = jnp.concatenate(
        [x, jnp.ones((1, N), jnp.float32)], axis=0)
    x_aug_bf = x_aug.astype(jnp.bfloat16)
    n_row = jnp.sum(p * p, axis=0, keepdims=True)
    ones_row = jnp.ones((1, N), jnp.float32)
    c_row = 0.25 * _RADIUS_SQ - 0.5 * n_row
    u = jnp.concatenate([p, c_row, ones_row], axis=0)
    v = jnp.concatenate([p, ones_row, c_row], axis=0)
    u_hi = u.astype(jnp.bfloat16)
    u_lo = (u - u_hi.astype(jnp.float32)).astype(jnp.bfloat16)
    v_hi = v.astype(jnp.bfloat16)
    v_lo = (v - v_hi.astype(jnp.float32)).astype(jnp.bfloat16)
    u_stack = jnp.concatenate([u_hi, u_hi, u_lo], axis=0)
    v_stack = jnp.concatenate([v_hi, v_lo, v_hi], axis=0)

    s_chunks = [jnp.zeros((C + 1, TJ), jnp.float32) for _ in range(NC)]
    for j in range(NC):
        slj = slice(j * TJ, (j + 1) * TJ)
        rows = (j + 1) * TJ
        s_j = _dot(u_stack[:, :rows], v_stack[:, slj], ((0,), (0,)),
                   lax.Precision.DEFAULT)
        mask = (s_j > 0.0).astype(jnp.float32).astype(jnp.bfloat16)
        contrib = _dot(x_aug_bf[:, slj], mask, ((1,), (1,)),
                       lax.Precision.DEFAULT)
        for i in range(j + 1):
            s_chunks[i] = s_chunks[i] + contrib[:, i * TJ:(i + 1) * TJ]
        if j > 0:
            s_chunks[j] = s_chunks[j] + _dot(
                x_aug_bf[:, :j * TJ], mask[:j * TJ], ((1,), (0,)),
                lax.Precision.DEFAULT)
    s = jnp.concatenate(s_chunks, axis=1)
    counts = jnp.maximum(s[C:], 1.0)
    neigh_mean = s[:C] / counts

    local_in = jnp.concatenate(
        [x, neigh_mean, jnp.broadcast_to(gu, (G, N))], axis=0)
    li_hi = local_in.astype(jnp.bfloat16)
    li_lo = (local_in - li_hi.astype(jnp.float32)).astype(jnp.bfloat16)
    li_stack = jnp.concatenate([li_hi, li_lo, li_hi], axis=0)
    W = WlT
    W_hi = W.astype(jnp.bfloat16)
    W_lo = (W - W_hi.astype(jnp.float32)).astype(jnp.bfloat16)
    W_stack = jnp.concatenate([W_hi, W_hi, W_lo], axis=1)
    lo = jnp.tanh(
        _dot(W_stack, li_stack, ((1,), (0,)), lax.Precision.DEFAULT)
        + bl)
    pos_out_ref[...] = lo[:P]
    feat_out_ref[...] = lo[P:]


def kernel(positions, features, global_features, W_g, b_g, W_l, b_l):
    B, P, N = positions.shape
    F = features.shape[1]
    C = P + F
    G = global_features.shape[1]

    x = jnp.concatenate([positions, features], axis=1)
    g3 = global_features[:, :, None]
    WgT = W_g.T
    bg = b_g[:, None]
    WlT = W_l.T
    bl = b_l[:, None]

    CB = 2 if B % 2 == 0 else 1
    pos_new, feat_new, gout = pl.pallas_call(
        functools.partial(_body, num_pos=P, num_glob=G),
        grid=(B // CB,),
        compiler_params=pltpu.CompilerParams(
            dimension_semantics=("parallel",)),
        in_specs=[
            pl.BlockSpec((CB, C, N), lambda b: (b, 0, 0)),
            pl.BlockSpec((CB, P, N), lambda b: (b, 0, 0)),
            pl.BlockSpec((CB, G, 1), lambda b: (b, 0, 0)),
            pl.BlockSpec((2 * G, C + G), lambda b: (0, 0)),
            pl.BlockSpec((2 * G, 1), lambda b: (0, 0)),
            pl.BlockSpec((C, 2 * C + G), lambda b: (0, 0)),
            pl.BlockSpec((C, 1), lambda b: (0, 0)),
        ],
        out_specs=[
            pl.BlockSpec((CB, P, N), lambda b: (b, 0, 0)),
            pl.BlockSpec((CB, F, N), lambda b: (b, 0, 0)),
            pl.BlockSpec((CB, 2 * G, 1), lambda b: (b, 0, 0)),
        ],
        out_shape=[
            jax.ShapeDtypeStruct((B, P, N), jnp.float32),
            jax.ShapeDtypeStruct((B, F, N), jnp.float32),
            jax.ShapeDtypeStruct((B, 2 * G, 1), jnp.float32),
        ],
    )(x, positions, g3, WgT, bg, WlT, bl)

    return (pos_new, feat_new, gout[:, :G, 0])


# speedup vs baseline: 1.1684x; 1.0029x over previous
import functools

import jax
import jax.numpy as jnp
from jax import lax
from jax.experimental import pallas as pl
from jax.experimental.pallas import tpu as pltpu

_RADIUS_SQ = 64.0
_TJ = 512


def _dot(a, b, dims, precision):
    return lax.dot_general(a, b, (dims, ((), ())),
                           preferred_element_type=jnp.float32,
                           precision=precision)


def _body(x_ref, p_ref, g_ref, WgT_ref, bg_ref, WlT_ref, bl_ref,
          pos_out_ref, feat_out_ref, g_out_ref, *, num_pos, num_glob):
    for c in range(x_ref.shape[0]):
        _one_cloud(x_ref[c], p_ref[c], g_ref[c], WgT_ref[...], bg_ref[...],
                   WlT_ref[...], bl_ref[...], pos_out_ref.at[c],
                   feat_out_ref.at[c], g_out_ref.at[c], num_pos, num_glob)


def _one_cloud(x, p, g_col, WgT, bg, WlT, bl,
               pos_out_ref, feat_out_ref, g_out_ref, num_pos, num_glob):
    P, G = num_pos, num_glob
    C, N = x.shape
    TJ = _TJ if N % _TJ == 0 else N
    NC = N // TJ

    agg = jnp.max(x, axis=1, keepdims=True)
    g_in = jnp.concatenate([agg, g_col], axis=0)
    g_out = jnp.tanh(
        _dot(WgT, g_in, ((1,), (0,)), lax.Precision.HIGHEST)
        + bg)
    g_out_ref[...] = g_out
    gu = g_out[G:]

    x_aug = jnp.concatenate(
        [x, jnp.ones((1, N), jnp.float32)], axis=0)
    x_aug_bf = x_aug.astype(jnp.bfloat16)
    n_row = jnp.sum(p * p, axis=0, keepdims=True)
    ones_row = jnp.ones((1, N), jnp.float32)
    c_row = 0.25 * _RADIUS_SQ - 0.5 * n_row
    u = jnp.concatenate([p, c_row, ones_row], axis=0)
    v = jnp.concatenate([p, ones_row, c_row], axis=0)
    u_hi = u.astype(jnp.bfloat16)
    u_lo = (u - u_hi.astype(jnp.float32)).astype(jnp.bfloat16)
    v_hi = v.astype(jnp.bfloat16)
    v_lo = (v - v_hi.astype(jnp.float32)).astype(jnp.bfloat16)
    u_stack = jnp.concatenate([u_hi, u_hi, u_lo], axis=0)
    v_stack = jnp.concatenate([v_hi, v_lo, v_hi], axis=0)

    s_chunks = [jnp.zeros((C + 1, TJ), jnp.float32) for _ in range(NC)]
    for j in range(NC):
        slj = slice(j * TJ, (j + 1) * TJ)
        rows = (j + 1) * TJ
        s_j = _dot(u_stack[:, :rows], v_stack[:, slj], ((0,), (0,)),
                   lax.Precision.DEFAULT)
        mask = (s_j > 0.0).astype(jnp.float32).astype(jnp.bfloat16)
        contrib = _dot(x_aug_bf[:, slj], mask, ((1,), (1,)),
                       lax.Precision.DEFAULT)
        for i in range(j + 1):
            s_chunks[i] = s_chunks[i] + contrib[:, i * TJ:(i + 1) * TJ]
        if j > 0:
            s_chunks[j] = s_chunks[j] + _dot(
                x_aug_bf[:, :j * TJ], mask[:j * TJ], ((1,), (0,)),
                lax.Precision.DEFAULT)
    s = jnp.concatenate(s_chunks, axis=1)
    counts = jnp.maximum(s[C:], 1.0)
    neigh_mean = s[:C] / counts

    local_in = jnp.concatenate(
        [x, neigh_mean, jnp.broadcast_to(gu, (G, N))], axis=0)
    li_hi = local_in.astype(jnp.bfloat16)
    li_lo = (local_in - li_hi.astype(jnp.float32)).astype(jnp.bfloat16)
    li_stack = jnp.concatenate([li_hi, li_lo, li_hi], axis=0)
    W = WlT
    W_hi = W.astype(jnp.bfloat16)
    W_lo = (W - W_hi.astype(jnp.float32)).astype(jnp.bfloat16)
    W_stack = jnp.concatenate([W_hi, W_hi, W_lo], axis=1)
    lo = jnp.tanh(
        _dot(W_stack, li_stack, ((1,), (0,)), lax.Precision.DEFAULT)
        + bl)
    pos_out_ref[...] = lo[:P]
    feat_out_ref[...] = lo[P:]


def kernel(positions, features, global_features, W_g, b_g, W_l, b_l):
    B, P, N = positions.shape
    F = features.shape[1]
    C = P + F
    G = global_features.shape[1]

    x = jnp.concatenate([positions, features], axis=1)
    g3 = global_features[:, :, None]
    WgT = W_g.T
    bg = b_g[:, None]
    WlT = W_l.T
    bl = b_l[:, None]

    CB = 4 if B % 4 == 0 else (2 if B % 2 == 0 else 1)
    pos_new, feat_new, gout = pl.pallas_call(
        functools.partial(_body, num_pos=P, num_glob=G),
        grid=(B // CB,),
        compiler_params=pltpu.CompilerParams(
            dimension_semantics=("parallel",)),
        in_specs=[
            pl.BlockSpec((CB, C, N), lambda b: (b, 0, 0)),
            pl.BlockSpec((CB, P, N), lambda b: (b, 0, 0)),
            pl.BlockSpec((CB, G, 1), lambda b: (b, 0, 0)),
            pl.BlockSpec((2 * G, C + G), lambda b: (0, 0)),
            pl.BlockSpec((2 * G, 1), lambda b: (0, 0)),
            pl.BlockSpec((C, 2 * C + G), lambda b: (0, 0)),
            pl.BlockSpec((C, 1), lambda b: (0, 0)),
        ],
        out_specs=[
            pl.BlockSpec((CB, P, N), lambda b: (b, 0, 0)),
            pl.BlockSpec((CB, F, N), lambda b: (b, 0, 0)),
            pl.BlockSpec((CB, 2 * G, 1), lambda b: (b, 0, 0)),
        ],
        out_shape=[
            jax.ShapeDtypeStruct((B, P, N), jnp.float32),
            jax.ShapeDtypeStruct((B, F, N), jnp.float32),
            jax.ShapeDtypeStruct((B, 2 * G, 1), jnp.float32),
        ],
    )(x, positions, g3, WgT, bg, WlT, bl)

    return (pos_new, feat_new, gout[:, :G, 0])
